# baseline (speedup 1.0000x reference)
import functools
import math

import jax
import jax.numpy as jnp
import numpy as np
from jax import lax
from jax.experimental import pallas as pl
from jax.experimental.pallas import tpu as pltpu
from jax.experimental.pallas import tpu_sc as plsc

VOCAB = 1000000
EMB = 64
B = 4096
S = 200
SCALE = math.sqrt(EMB)

_info = plsc.get_sparse_core_info()
NC, NS, L = _info.num_cores, _info.num_subcores, _info.num_lanes
NW = NC * NS
SEQ_PER_W = B // NW
N_VREG = EMB // L


def _pos_encoding_np(max_len, d):
    pos = np.arange(max_len)[:, None].astype(np.float32)
    div = np.exp(np.arange(0, d, 2).astype(np.float32) * (-math.log(10000.0) / d))
    pe = np.zeros((max_len, d), dtype=np.float32)
    pe[:, 0::2] = np.sin(pos * div)
    pe[:, 1::2] = np.cos(pos * div)
    return pe


_PE_NP = _pos_encoding_np(S, EMB)


def _body(tok_hbm, idx_hbm, pe_hbm, out_hbm, pe_v, idx_v, rows_v, sem):
    wid = lax.axis_index("s") * NC + lax.axis_index("c")
    seq0 = wid * SEQ_PER_W

    pltpu.sync_copy(pe_hbm, pe_v)

    def step(i, carry):
        r0 = (seq0 + i) * S
        pltpu.sync_copy(idx_hbm.at[pl.ds(r0, S)], idx_v)
        cp1 = pltpu.async_copy(
            tok_hbm.at[idx_v.at[pl.ds(0, 128)]], rows_v.at[pl.ds(0, 128)], sem
        )
        cp2 = pltpu.async_copy(
            tok_hbm.at[idx_v.at[pl.ds(128, S - 128)]],
            rows_v.at[pl.ds(128, S - 128)],
            sem,
        )
        cp1.wait()
        cp2.wait()

        def crow(r, c):
            for j in range(N_VREG):
                sl = pl.ds(j * L, L)
                rows_v[r, sl] = rows_v[r, sl] * SCALE + pe_v[r, sl]
            return c

        lax.fori_loop(0, S, crow, 0)
        pltpu.sync_copy(rows_v, out_hbm.at[pl.ds(r0, S)])
        return carry

    lax.fori_loop(0, SEQ_PER_W, step, 0)


@functools.partial(jax.jit, static_argnames=())
def _emb_lookup(tok_emb, xf, pe):
    mesh = plsc.VectorSubcoreMesh(core_axis_name="c", subcore_axis_name="s")
    f = pl.kernel(
        _body,
        mesh=mesh,
        out_type=jax.ShapeDtypeStruct((B * S, EMB), jnp.float32),
        scratch_types=[
            pltpu.VMEM((S, EMB), jnp.float32),
            pltpu.VMEM((S,), jnp.int32),
            pltpu.VMEM((S, EMB), jnp.float32),
            pltpu.SemaphoreType.DMA,
        ],
        compiler_params=pltpu.CompilerParams(use_tc_tiling_on_sc=False),
    )
    return f(tok_emb, xf, pe)


def kernel(x, tok_emb):
    xf = x.reshape(-1).astype(jnp.int32)
    pe = jnp.asarray(_PE_NP)
    out = _emb_lookup(tok_emb, xf, pe)
    return out.reshape(B, S, EMB)


# speedup vs baseline: 1.1855x; 1.1855x over previous
import functools
import math

import jax
import jax.numpy as jnp
import numpy as np
from jax import lax
from jax.experimental import pallas as pl
from jax.experimental.pallas import tpu as pltpu
from jax.experimental.pallas import tpu_sc as plsc

VOCAB = 1000000
EMB = 64
B = 4096
S = 200
SCALE = math.sqrt(EMB)

_info = plsc.get_sparse_core_info()
NC, NS, L = _info.num_cores, _info.num_subcores, _info.num_lanes
NW = NC * NS
SEQ_PER_W = B // NW
N_VREG = EMB // L
G1 = 128
G2 = S - G1


def _pos_encoding_np(max_len, d):
    pos = np.arange(max_len)[:, None].astype(np.float32)
    div = np.exp(np.arange(0, d, 2).astype(np.float32) * (-math.log(10000.0) / d))
    pe = np.zeros((max_len, d), dtype=np.float32)
    pe[:, 0::2] = np.sin(pos * div)
    pe[:, 1::2] = np.cos(pos * div)
    return pe


_PE_NP = _pos_encoding_np(S, EMB)


def _body(tok_hbm, x_hbm, pe_hbm, out_hbm, pe_v, idx_v, rows, gsems, ssems):
    wid = lax.axis_index("s") * NC + lax.axis_index("c")
    seq0 = wid * SEQ_PER_W

    pltpu.sync_copy(pe_hbm, pe_v)
    pltpu.sync_copy(x_hbm.at[pl.ds(seq0, SEQ_PER_W)], idx_v)

    def fire_gather(i, p):
        pltpu.async_copy(
            tok_hbm.at[idx_v.at[i, pl.ds(0, G1)]], rows[p].at[pl.ds(0, G1)], gsems[p]
        )
        pltpu.async_copy(
            tok_hbm.at[idx_v.at[i, pl.ds(G1, G2)]], rows[p].at[pl.ds(G1, G2)], gsems[p]
        )

    def wait_gather(i, p):
        pltpu.make_async_copy(
            tok_hbm.at[idx_v.at[i, pl.ds(0, G1)]], rows[p].at[pl.ds(0, G1)], gsems[p]
        ).wait()
        pltpu.make_async_copy(
            tok_hbm.at[idx_v.at[i, pl.ds(G1, G2)]], rows[p].at[pl.ds(G1, G2)], gsems[p]
        ).wait()

    def fire_scatter(i, p):
        pltpu.async_copy(rows[p], out_hbm.at[seq0 + i], ssems[p])

    def wait_scatter(i, p):
        pltpu.make_async_copy(rows[p], out_hbm.at[seq0 + i], ssems[p]).wait()

    def compute(p):
        rv = rows[p]

        def crow(r, c):
            for u in range(2):
                for j in range(N_VREG):
                    sl = pl.ds(j * L, L)
                    rv[2 * r + u, sl] = rv[2 * r + u, sl] * SCALE + pe_v[2 * r + u, sl]
            return c

        lax.fori_loop(0, S // 2, crow, 0)

    def step(i, p, first=False, last=False):
        if not first:
            wait_scatter(i - 1, 1 - p)
        if not last:
            fire_gather(i + 1, 1 - p)
        wait_gather(i, p)
        compute(p)
        fire_scatter(i, p)

    fire_gather(0, 0)
    step(0, 0, first=True)

    def pair(k, c):
        step(2 * k + 1, 1)
        step(2 * k + 2, 0)
        return c

    lax.fori_loop(0, (SEQ_PER_W - 2) // 2, pair, 0)
    step(SEQ_PER_W - 1, 1, last=True)
    wait_scatter(SEQ_PER_W - 1, 1)


@jax.jit
def _emb_lookup(tok_emb, x, pe):
    mesh = plsc.VectorSubcoreMesh(core_axis_name="c", subcore_axis_name="s")
    f = pl.kernel(
        _body,
        mesh=mesh,
        out_type=jax.ShapeDtypeStruct((B, S, EMB), jnp.float32),
        scratch_types=[
            pltpu.VMEM((S, EMB), jnp.float32),
            pltpu.VMEM((SEQ_PER_W, S), jnp.int32),
            [pltpu.VMEM((S, EMB), jnp.float32) for _ in range(2)],
            [pltpu.SemaphoreType.DMA for _ in range(2)],
            [pltpu.SemaphoreType.DMA for _ in range(2)],
        ],
        compiler_params=pltpu.CompilerParams(use_tc_tiling_on_sc=False),
    )
    return f(tok_emb, x, pe)


def kernel(x, tok_emb):
    pe = jnp.asarray(_PE_NP)
    return _emb_lookup(tok_emb, x.astype(jnp.int32), pe)
